# initial kernel scaffold (unmeasured)
import jax
import jax.numpy as jnp
from jax import lax
from jax.experimental import pallas as pl
from jax.experimental.pallas import tpu as pltpu

N_DEV = 8
N_TOK = 2048
D = 1024
N_EXP = 64
E_LOCAL = N_EXP // N_DEV
CHUNK = N_TOK // N_DEV


def kernel(x, router_W, route_idx, expert_W, shared_W):
    def body(x_ref, rw_ref, idx_ref, ew_ref, sw_ref, out_ref,
             partial_ref, send_ref, recv_ref, send_sem, recv_sem,
             credit_sem):
        my = lax.axis_index("i")
        left = lax.rem(my + N_DEV - 1, N_DEV)
        right = lax.rem(my + 1, N_DEV)

        barrier = pltpu.get_barrier_semaphore()
        for nbr in (left, right):
            pl.semaphore_signal(
                barrier, inc=1,
                device_id=(nbr,), device_id_type=pl.DeviceIdType.MESH,
            )
        pl.semaphore_wait(barrier, 2)

        xv = x_ref[...]
        scores = jnp.dot(xv, rw_ref[...], preferred_element_type=jnp.float32)
        m = jnp.max(scores, axis=-1, keepdims=True)
        p = jnp.exp(scores - m)
        probs = p / jnp.sum(p, axis=-1, keepdims=True)
        idx = idx_ref[...]
        eids = lax.broadcasted_iota(jnp.int32, (N_TOK, N_EXP), 1)
        gate = jnp.sum(
            jnp.where(idx == eids, probs, 0.0), axis=-1, keepdims=True
        )

        acc = jnp.zeros((N_TOK, D), jnp.float32)
        for k in range(E_LOCAL):
            e = my * E_LOCAL + k
            coeff = jnp.where(idx == e, gate, 0.0)
            acc = acc + coeff * jnp.dot(
                xv, ew_ref[k], preferred_element_type=jnp.float32
            )
        partial_ref[...] = acc

        for s in range(N_DEV - 1):
            c = lax.rem(my + 2 * N_DEV - 1 - s, N_DEV)
            chunk = partial_ref[pl.ds(c * CHUNK, CHUNK), :]
            if s == 0:
                send_ref[...] = chunk
            else:
                send_ref[...] = recv_ref[...] + chunk
                pl.semaphore_signal(
                    credit_sem, inc=1,
                    device_id=(left,), device_id_type=pl.DeviceIdType.MESH,
                )
                pl.semaphore_wait(credit_sem, 1)
            rdma = pltpu.make_async_remote_copy(
                src_ref=send_ref,
                dst_ref=recv_ref,
                send_sem=send_sem,
                recv_sem=recv_sem,
                device_id=(right,),
                device_id_type=pl.DeviceIdType.MESH,
            )
            rdma.start()
            rdma.wait()

        my_rows = pl.ds(my * CHUNK, CHUNK)
        shared_part = jnp.dot(
            x_ref[my_rows, :], sw_ref[...], preferred_element_type=jnp.float32
        )
        out_ref[...] = recv_ref[...] + partial_ref[my_rows, :] + shared_part

    return pl.pallas_call(
        body,
        out_shape=jax.ShapeDtypeStruct((CHUNK, D), jnp.float32),
        in_specs=[pl.BlockSpec(memory_space=pltpu.VMEM)] * 5,
        out_specs=pl.BlockSpec(memory_space=pltpu.VMEM),
        scratch_shapes=[
            pltpu.VMEM((N_TOK, D), jnp.float32),
            pltpu.VMEM((CHUNK, D), jnp.float32),
            pltpu.VMEM((CHUNK, D), jnp.float32),
            pltpu.SemaphoreType.DMA,
            pltpu.SemaphoreType.DMA,
            pltpu.SemaphoreType.REGULAR,
        ],
        compiler_params=pltpu.CompilerParams(collective_id=0),
    )(x, router_W, route_idx, expert_W, shared_W)


# baseline (device time: 175369 ns/iter reference)
import jax
import jax.numpy as jnp
from jax import lax
from jax.experimental import pallas as pl
from jax.experimental.pallas import tpu as pltpu

N_DEV = 8
N_TOK = 2048
D = 1024
N_EXP = 64
E_LOCAL = N_EXP // N_DEV
CHUNK = N_TOK // N_DEV
HALF = N_TOK // 2


def kernel(x, router_W, route_idx, expert_W, shared_W):
    def body(x_ref, rw_ref, idx_ref, ew_ref, sw_ref, out_ref,
             partial_ref, w_ref, send_ref, recv_ref,
             copy_sem, send_sem, recv_sem, credit_sem):
        my = lax.axis_index("i")
        left = lax.rem(my + N_DEV - 1, N_DEV)
        right = lax.rem(my + 1, N_DEV)

        barrier = pltpu.get_barrier_semaphore()
        for nbr in (left, right):
            pl.semaphore_signal(
                barrier, inc=1,
                device_id=(nbr,), device_id_type=pl.DeviceIdType.MESH,
            )
        pl.semaphore_wait(barrier, 2)

        scores = jnp.dot(
            x_ref[...], rw_ref[...], preferred_element_type=jnp.float32
        )
        m = jnp.max(scores, axis=-1, keepdims=True)
        p = jnp.exp(scores - m)
        probs = p / jnp.sum(p, axis=-1, keepdims=True)
        idx = idx_ref[...]
        eids = lax.broadcasted_iota(jnp.int32, (N_TOK, N_EXP), 1)
        gate = jnp.sum(
            jnp.where(idx == eids, probs, 0.0), axis=-1, keepdims=True
        )

        for k in range(E_LOCAL):
            cp = pltpu.make_async_copy(ew_ref.at[k], w_ref, copy_sem)
            cp.start()
            cp.wait()
            e = my * E_LOCAL + k
            coeff = jnp.where(idx == e, gate, 0.0)
            for h in range(2):
                rows = pl.ds(h * HALF, HALF)
                t = coeff[h * HALF:(h + 1) * HALF] * jnp.dot(
                    x_ref[rows, :], w_ref[...],
                    preferred_element_type=jnp.float32,
                )
                if k == 0:
                    partial_ref[rows, :] = t
                else:
                    partial_ref[rows, :] = partial_ref[rows, :] + t

        cp = pltpu.make_async_copy(sw_ref, w_ref, copy_sem)
        cp.start()
        cp.wait()
        my_rows = pl.ds(my * CHUNK, CHUNK)
        partial_ref[my_rows, :] = partial_ref[my_rows, :] + jnp.dot(
            x_ref[my_rows, :], w_ref[...], preferred_element_type=jnp.float32
        )

        for s in range(N_DEV - 1):
            c = lax.rem(my + 2 * N_DEV - 1 - s, N_DEV)
            chunk = partial_ref[pl.ds(c * CHUNK, CHUNK), :]
            if s == 0:
                send_ref[...] = chunk
            else:
                send_ref[...] = recv_ref[...] + chunk
                pl.semaphore_signal(
                    credit_sem, inc=1,
                    device_id=(left,), device_id_type=pl.DeviceIdType.MESH,
                )
                pl.semaphore_wait(credit_sem, 1)
            rdma = pltpu.make_async_remote_copy(
                src_ref=send_ref,
                dst_ref=recv_ref,
                send_sem=send_sem,
                recv_sem=recv_sem,
                device_id=(right,),
                device_id_type=pl.DeviceIdType.MESH,
            )
            rdma.start()
            rdma.wait()

        out_ref[...] = recv_ref[...] + partial_ref[my_rows, :]

    return pl.pallas_call(
        body,
        out_shape=jax.ShapeDtypeStruct((CHUNK, D), jnp.float32),
        in_specs=[
            pl.BlockSpec(memory_space=pltpu.VMEM),
            pl.BlockSpec(memory_space=pltpu.VMEM),
            pl.BlockSpec(memory_space=pltpu.VMEM),
            pl.BlockSpec(memory_space=pl.ANY),
            pl.BlockSpec(memory_space=pl.ANY),
        ],
        out_specs=pl.BlockSpec(memory_space=pltpu.VMEM),
        scratch_shapes=[
            pltpu.VMEM((N_TOK, D), jnp.float32),
            pltpu.VMEM((D, D), jnp.float32),
            pltpu.VMEM((CHUNK, D), jnp.float32),
            pltpu.VMEM((CHUNK, D), jnp.float32),
            pltpu.SemaphoreType.DMA,
            pltpu.SemaphoreType.DMA,
            pltpu.SemaphoreType.DMA,
            pltpu.SemaphoreType.REGULAR,
        ],
        compiler_params=pltpu.CompilerParams(collective_id=0),
    )(x, router_W, route_idx, expert_W, shared_W)


# device time: 73956 ns/iter; 2.3713x vs baseline; 2.3713x over previous
import jax
import jax.numpy as jnp
from jax import lax
from jax.experimental import pallas as pl
from jax.experimental.pallas import tpu as pltpu

import os

_ABLATE_NO_RING = os.environ.get("ABLATE_NO_RING") == "1"

N_DEV = 8
N_TOK = 2048
D = 1024
N_EXP = 64
E_LOCAL = N_EXP // N_DEV
CHUNK = N_TOK // N_DEV
HALF = N_TOK // 2


def kernel(x, router_W, route_idx, expert_W, shared_W):
    def body(x_ref, rw_ref, idx_ref, ew_ref, sw_ref, out_ref,
             partial_ref, w_ref, send_ref, recv_ref,
             copy_sem, send_sem, recv_sem, credit_sem):
        my = lax.axis_index("i")
        left = lax.rem(my + N_DEV - 1, N_DEV)
        right = lax.rem(my + 1, N_DEV)

        barrier = pltpu.get_barrier_semaphore()
        for nbr in (left, right):
            pl.semaphore_signal(
                barrier, inc=1,
                device_id=(nbr,), device_id_type=pl.DeviceIdType.MESH,
            )
        pl.semaphore_wait(barrier, 2)

        scores = jnp.dot(
            x_ref[...], rw_ref[...], preferred_element_type=jnp.float32
        )
        m = jnp.max(scores, axis=-1, keepdims=True)
        p = jnp.exp(scores - m)
        probs = p / jnp.sum(p, axis=-1, keepdims=True)
        idx = idx_ref[...]
        eids = lax.broadcasted_iota(jnp.int32, (N_TOK, N_EXP), 1)
        gate = jnp.sum(
            jnp.where(idx == eids, probs, 0.0), axis=-1, keepdims=True
        )

        for k in range(E_LOCAL):
            cp = pltpu.make_async_copy(ew_ref.at[k], w_ref, copy_sem)
            cp.start()
            cp.wait()
            e = my * E_LOCAL + k
            coeff = jnp.where(idx == e, gate, 0.0)
            for h in range(2):
                rows = pl.ds(h * HALF, HALF)
                t = coeff[h * HALF:(h + 1) * HALF] * jnp.dot(
                    x_ref[rows, :], w_ref[...],
                    preferred_element_type=jnp.float32,
                )
                if k == 0:
                    partial_ref[rows, :] = t
                else:
                    partial_ref[rows, :] = partial_ref[rows, :] + t

        cp = pltpu.make_async_copy(sw_ref, w_ref, copy_sem)
        cp.start()
        cp.wait()
        my_rows = pl.ds(my * CHUNK, CHUNK)
        partial_ref[my_rows, :] = partial_ref[my_rows, :] + jnp.dot(
            x_ref[my_rows, :], w_ref[...], preferred_element_type=jnp.float32
        )

        if _ABLATE_NO_RING:
            out_ref[...] = partial_ref[my_rows, :]
            return

        for s in range(N_DEV - 1):
            c = lax.rem(my + 2 * N_DEV - 1 - s, N_DEV)
            chunk = partial_ref[pl.ds(c * CHUNK, CHUNK), :]
            if s == 0:
                send_ref[...] = chunk
            else:
                send_ref[...] = recv_ref[...] + chunk
                pl.semaphore_signal(
                    credit_sem, inc=1,
                    device_id=(left,), device_id_type=pl.DeviceIdType.MESH,
                )
                pl.semaphore_wait(credit_sem, 1)
            rdma = pltpu.make_async_remote_copy(
                src_ref=send_ref,
                dst_ref=recv_ref,
                send_sem=send_sem,
                recv_sem=recv_sem,
                device_id=(right,),
                device_id_type=pl.DeviceIdType.MESH,
            )
            rdma.start()
            rdma.wait()

        out_ref[...] = recv_ref[...] + partial_ref[my_rows, :]

    return pl.pallas_call(
        body,
        out_shape=jax.ShapeDtypeStruct((CHUNK, D), jnp.float32),
        in_specs=[
            pl.BlockSpec(memory_space=pltpu.VMEM),
            pl.BlockSpec(memory_space=pltpu.VMEM),
            pl.BlockSpec(memory_space=pltpu.VMEM),
            pl.BlockSpec(memory_space=pl.ANY),
            pl.BlockSpec(memory_space=pl.ANY),
        ],
        out_specs=pl.BlockSpec(memory_space=pltpu.VMEM),
        scratch_shapes=[
            pltpu.VMEM((N_TOK, D), jnp.float32),
            pltpu.VMEM((D, D), jnp.float32),
            pltpu.VMEM((CHUNK, D), jnp.float32),
            pltpu.VMEM((CHUNK, D), jnp.float32),
            pltpu.SemaphoreType.DMA,
            pltpu.SemaphoreType.DMA,
            pltpu.SemaphoreType.DMA,
            pltpu.SemaphoreType.REGULAR,
        ],
        compiler_params=pltpu.CompilerParams(collective_id=0),
    )(x, router_W, route_idx, expert_W, shared_W)


# device time: 62062 ns/iter; 2.8257x vs baseline; 1.1916x over previous
import jax
import jax.numpy as jnp
from jax import lax
from jax.experimental import pallas as pl
from jax.experimental.pallas import tpu as pltpu

N_DEV = 8
N_TOK = 2048
D = 1024
N_EXP = 64
E_LOCAL = N_EXP // N_DEV
CHUNK = N_TOK // N_DEV
CAP = 64


def _expert_gemms(xg, cg, expert_W, shared_W, xmine):

    def body(xg_ref, cg_ref, xm_ref, ew_ref, sw_ref, yg_ref, ys_ref,
             w_ref, sems):
        pltpu.make_async_copy(sw_ref, w_ref.at[2], sems.at[2]).start()
        pltpu.make_async_copy(ew_ref.at[0], w_ref.at[0], sems.at[0]).start()
        for k in range(E_LOCAL):
            if k + 1 < E_LOCAL:
                pltpu.make_async_copy(
                    ew_ref.at[k + 1], w_ref.at[(k + 1) % 2],
                    sems.at[(k + 1) % 2],
                ).start()
            pltpu.make_async_copy(
                ew_ref.at[k], w_ref.at[k % 2], sems.at[k % 2]
            ).wait()
            yg_ref[k] = cg_ref[k] * jnp.dot(
                xg_ref[k], w_ref[k % 2], preferred_element_type=jnp.float32
            )
        pltpu.make_async_copy(sw_ref, w_ref.at[2], sems.at[2]).wait()
        ys_ref[...] = jnp.dot(
            xm_ref[...], w_ref[2], preferred_element_type=jnp.float32
        )

    return pl.pallas_call(
        body,
        out_shape=(
            jax.ShapeDtypeStruct((E_LOCAL, CAP, D), jnp.float32),
            jax.ShapeDtypeStruct((CHUNK, D), jnp.float32),
        ),
        in_specs=[
            pl.BlockSpec(memory_space=pltpu.VMEM),
            pl.BlockSpec(memory_space=pltpu.VMEM),
            pl.BlockSpec(memory_space=pltpu.VMEM),
            pl.BlockSpec(memory_space=pl.ANY),
            pl.BlockSpec(memory_space=pl.ANY),
        ],
        out_specs=(
            pl.BlockSpec(memory_space=pltpu.VMEM),
            pl.BlockSpec(memory_space=pltpu.VMEM),
        ),
        scratch_shapes=[
            pltpu.VMEM((3, D, D), jnp.float32),
            pltpu.SemaphoreType.DMA((3,)),
        ],
    )(xg, cg, xmine, expert_W, shared_W)


def _all_to_all_combine(sbufs, P, ys):

    def body(sb_ref, p_ref, ys_ref, out_ref, recv_ref, send_sems,
             recv_sems):
        my = lax.axis_index("i")
        barrier = pltpu.get_barrier_semaphore()
        for o in range(1, N_DEV):
            pl.semaphore_signal(
                barrier, inc=1,
                device_id=(lax.rem(my + o, N_DEV),),
                device_id_type=pl.DeviceIdType.MESH,
            )
        pl.semaphore_wait(barrier, N_DEV - 1)

        def rdma(o):
            return pltpu.make_async_remote_copy(
                src_ref=sb_ref.at[o],
                dst_ref=recv_ref.at[N_DEV - o],
                send_sem=send_sems.at[o],
                recv_sem=recv_sems.at[N_DEV - o],
                device_id=(lax.rem(my + o, N_DEV),),
                device_id_type=pl.DeviceIdType.MESH,
            )

        for o in range(1, N_DEV):
            rdma(o).start()

        acc = ys_ref[...] + jnp.dot(
            p_ref[0], sb_ref[0], preferred_element_type=jnp.float32
        )
        for s in range(1, N_DEV):
            rdma(N_DEV - s).wait_recv()
            acc = acc + jnp.dot(
                p_ref[s], recv_ref[s], preferred_element_type=jnp.float32
            )
        out_ref[...] = acc
        for o in range(1, N_DEV):
            rdma(o).wait_send()

    return pl.pallas_call(
        body,
        out_shape=jax.ShapeDtypeStruct((CHUNK, D), jnp.float32),
        in_specs=[
            pl.BlockSpec(memory_space=pltpu.VMEM),
            pl.BlockSpec(memory_space=pltpu.VMEM),
            pl.BlockSpec(memory_space=pltpu.VMEM),
        ],
        out_specs=pl.BlockSpec(memory_space=pltpu.VMEM),
        scratch_shapes=[
            pltpu.VMEM((N_DEV, CAP, D), jnp.float32),
            pltpu.SemaphoreType.DMA((N_DEV,)),
            pltpu.SemaphoreType.DMA((N_DEV,)),
        ],
        compiler_params=pltpu.CompilerParams(collective_id=0),
    )(sbufs, P, ys)


def kernel(x, router_W, route_idx, expert_W, shared_W):
    my = lax.axis_index("i")
    idxv = route_idx[:, 0]
    owner = idxv // E_LOCAL

    scores = x @ router_W
    m = jnp.max(scores, axis=-1, keepdims=True)
    p = jnp.exp(scores - m)
    probs = p / jnp.sum(p, axis=-1, keepdims=True)
    gate = jnp.take_along_axis(probs, route_idx, axis=1)[:, 0]

    my_eids = my * E_LOCAL + jnp.arange(E_LOCAL, dtype=jnp.int32)
    condk = idxv[None, :] == my_eids[:, None]
    rank = jnp.cumsum(condk, axis=1, dtype=jnp.int32) - 1
    counts = jnp.sum(condk, axis=1, dtype=jnp.int32)
    sel = condk[:, None, :] & (
        rank[:, None, :] == jnp.arange(CAP, dtype=jnp.int32)[None, :, None]
    )
    rows = jnp.arange(N_TOK, dtype=jnp.int32)
    ids = jnp.sum(sel * rows[None, None, :], axis=-1, dtype=jnp.int32)
    validg = jnp.arange(CAP, dtype=jnp.int32)[None, :] < counts[:, None]
    xg = jnp.take(x, ids.reshape(-1), axis=0, mode="clip").reshape(
        E_LOCAL, CAP, D
    )
    cg = (
        jnp.take(gate, ids.reshape(-1), mode="clip").reshape(E_LOCAL, CAP)
        * validg.astype(jnp.float32)
    )[..., None]
    xmine = lax.dynamic_slice(x, (my * CHUNK, 0), (CHUNK, D))

    yg, ys = _expert_gemms(xg, cg, expert_W, shared_W, xmine)

    rid = ids.reshape(-1)
    validf = validg.reshape(-1)
    dd = rid // CHUNK
    offs = jnp.arange(N_DEV, dtype=jnp.int32)
    cs = jnp.mod(my + offs, N_DEV)
    conds = validf[None, :] & (dd[None, :] == cs[:, None])
    before = (rid[None, :] < rid[:, None]).astype(jnp.float32)
    ranks = jnp.einsum(
        "oj,ij->oi", conds.astype(jnp.float32), before
    ).astype(jnp.int32)
    G = (
        conds[:, None, :]
        & (ranks[:, None, :] == jnp.arange(CAP, dtype=jnp.int32)[None, :, None])
    ).astype(jnp.float32)
    sbufs = jnp.einsum("oji,id->ojd", G, yg.reshape(N_DEV * CAP, D))

    owner_my = lax.dynamic_slice(owner, (my * CHUNK,), (CHUNK,))
    srcs = jnp.mod(my + offs, N_DEV)
    condr = owner_my[None, :] == srcs[:, None]
    jr = jnp.cumsum(condr, axis=1, dtype=jnp.int32) - 1
    P = (
        condr[:, :, None]
        & (jr[:, :, None] == jnp.arange(CAP, dtype=jnp.int32)[None, None, :])
    ).astype(jnp.float32)

    return _all_to_all_combine(sbufs, P, ys)


# device time: 58872 ns/iter; 2.9788x vs baseline; 1.0542x over previous
import jax
import jax.numpy as jnp
from jax import lax
from jax.experimental import pallas as pl
from jax.experimental.pallas import tpu as pltpu

N_DEV = 8
N_TOK = 2048
D = 1024
N_EXP = 64
E_LOCAL = N_EXP // N_DEV
CHUNK = N_TOK // N_DEV
CAP = 64


def _expert_gemms(x, router_W, oh, sel, expert_W, shared_W):

    def body(x_ref, rw_ref, oh_ref, sel_ref, ew_ref, sw_ref,
             yg_ref, ys_ref, w_ref, sems):
        my = lax.axis_index("i")
        pltpu.make_async_copy(ew_ref.at[0], w_ref.at[0], sems.at[0]).start()
        pltpu.make_async_copy(ew_ref.at[1], w_ref.at[1], sems.at[1]).start()

        scores = jnp.dot(
            x_ref[...], rw_ref[...], preferred_element_type=jnp.float32
        )
        m = jnp.max(scores, axis=-1, keepdims=True)
        e = jnp.exp(scores - m)
        probs = e / jnp.sum(e, axis=-1, keepdims=True)
        gate = jnp.sum(probs * oh_ref[...], axis=-1, keepdims=True)
        cg = jnp.dot(
            sel_ref[...], gate, preferred_element_type=jnp.float32
        )

        for k in range(E_LOCAL):
            pltpu.make_async_copy(
                ew_ref.at[k], w_ref.at[k % 2], sems.at[k % 2]
            ).wait()
            xg = jnp.dot(
                sel_ref[pl.ds(k * CAP, CAP), :], x_ref[...],
                preferred_element_type=jnp.float32,
            )
            yg_ref[pl.ds(k * CAP, CAP), :] = cg[k * CAP:(k + 1) * CAP] * jnp.dot(
                xg, w_ref[k % 2], preferred_element_type=jnp.float32
            )
            if k + 2 < E_LOCAL:
                pltpu.make_async_copy(
                    ew_ref.at[k + 2], w_ref.at[k % 2], sems.at[k % 2]
                ).start()
            elif k == E_LOCAL - 2:
                pltpu.make_async_copy(
                    sw_ref, w_ref.at[0], sems.at[0]
                ).start()
        pltpu.make_async_copy(sw_ref, w_ref.at[0], sems.at[0]).wait()
        ys_ref[...] = jnp.dot(
            x_ref[pl.ds(my * CHUNK, CHUNK), :], w_ref[0],
            preferred_element_type=jnp.float32,
        )

    return pl.pallas_call(
        body,
        out_shape=(
            jax.ShapeDtypeStruct((E_LOCAL * CAP, D), jnp.float32),
            jax.ShapeDtypeStruct((CHUNK, D), jnp.float32),
        ),
        in_specs=[
            pl.BlockSpec(memory_space=pltpu.VMEM),
            pl.BlockSpec(memory_space=pltpu.VMEM),
            pl.BlockSpec(memory_space=pltpu.VMEM),
            pl.BlockSpec(memory_space=pltpu.VMEM),
            pl.BlockSpec(memory_space=pl.ANY),
            pl.BlockSpec(memory_space=pl.ANY),
        ],
        out_specs=(
            pl.BlockSpec(memory_space=pltpu.VMEM),
            pl.BlockSpec(memory_space=pltpu.VMEM),
        ),
        scratch_shapes=[
            pltpu.VMEM((2, D, D), jnp.float32),
            pltpu.SemaphoreType.DMA((2,)),
        ],
    )(x, router_W, oh, sel, expert_W, shared_W)


def _all_to_all_combine(yg, G, P, ys):

    def body(yg_ref, g_ref, p_ref, ys_ref, out_ref, sb_ref, recv_ref,
             send_sems, recv_sems):
        my = lax.axis_index("i")
        barrier = pltpu.get_barrier_semaphore()
        for o in range(1, N_DEV):
            pl.semaphore_signal(
                barrier, inc=1,
                device_id=(lax.rem(my + o, N_DEV),),
                device_id_type=pl.DeviceIdType.MESH,
            )
        pl.semaphore_wait(barrier, N_DEV - 1)

        def rdma(o):
            return pltpu.make_async_remote_copy(
                src_ref=sb_ref.at[o],
                dst_ref=recv_ref.at[N_DEV - o],
                send_sem=send_sems.at[o],
                recv_sem=recv_sems.at[N_DEV - o],
                device_id=(lax.rem(my + o, N_DEV),),
                device_id_type=pl.DeviceIdType.MESH,
            )

        for o in range(1, N_DEV):
            sb_ref[o] = jnp.dot(
                g_ref[o], yg_ref[...], preferred_element_type=jnp.float32
            )
            rdma(o).start()

        sb_ref[0] = jnp.dot(
            g_ref[0], yg_ref[...], preferred_element_type=jnp.float32
        )
        acc = ys_ref[...] + jnp.dot(
            p_ref[0], sb_ref[0], preferred_element_type=jnp.float32
        )
        for s in range(1, N_DEV):
            rdma(N_DEV - s).wait_recv()
            acc = acc + jnp.dot(
                p_ref[s], recv_ref[s], preferred_element_type=jnp.float32
            )
        out_ref[...] = acc
        for o in range(1, N_DEV):
            rdma(o).wait_send()

    return pl.pallas_call(
        body,
        out_shape=jax.ShapeDtypeStruct((CHUNK, D), jnp.float32),
        in_specs=[
            pl.BlockSpec(memory_space=pltpu.VMEM),
            pl.BlockSpec(memory_space=pltpu.VMEM),
            pl.BlockSpec(memory_space=pltpu.VMEM),
            pl.BlockSpec(memory_space=pltpu.VMEM),
        ],
        out_specs=pl.BlockSpec(memory_space=pltpu.VMEM),
        scratch_shapes=[
            pltpu.VMEM((N_DEV, CAP, D), jnp.float32),
            pltpu.VMEM((N_DEV, CAP, D), jnp.float32),
            pltpu.SemaphoreType.DMA((N_DEV,)),
            pltpu.SemaphoreType.DMA((N_DEV,)),
        ],
        compiler_params=pltpu.CompilerParams(collective_id=0),
    )(yg, G, P, ys)


def kernel(x, router_W, route_idx, expert_W, shared_W):
    my = lax.axis_index("i")
    idxv = route_idx[:, 0]
    owner = idxv // E_LOCAL
    caps = jnp.arange(CAP, dtype=jnp.int32)
    rows = jnp.arange(N_TOK, dtype=jnp.int32)

    oh = (idxv[:, None] == jnp.arange(N_EXP, dtype=jnp.int32)[None, :]
          ).astype(jnp.float32)

    my_eids = my * E_LOCAL + jnp.arange(E_LOCAL, dtype=jnp.int32)
    condk = idxv[None, :] == my_eids[:, None]
    rank = jnp.cumsum(condk, axis=1, dtype=jnp.int32) - 1
    counts = jnp.sum(condk, axis=1, dtype=jnp.int32)
    selb = condk[:, None, :] & (rank[:, None, :] == caps[None, :, None])
    sel = selb.reshape(E_LOCAL * CAP, N_TOK).astype(jnp.float32)
    ids = jnp.sum(
        selb * rows[None, None, :], axis=-1, dtype=jnp.int32
    ).reshape(-1)
    validf = (caps[None, :] < counts[:, None]).reshape(-1)

    yg, ys = _expert_gemms(x, router_W, oh, sel, expert_W, shared_W)

    dd = ids // CHUNK
    offs = jnp.arange(N_DEV, dtype=jnp.int32)
    cs = jnp.mod(my + offs, N_DEV)
    conds = validf[None, :] & (dd[None, :] == cs[:, None])
    before = (ids[None, :] < ids[:, None]).astype(jnp.float32)
    ranks = jnp.einsum(
        "oj,ij->oi", conds.astype(jnp.float32), before
    ).astype(jnp.int32)
    G = (
        conds[:, None, :] & (ranks[:, None, :] == caps[None, :, None])
    ).astype(jnp.float32)

    owner_my = lax.dynamic_slice(owner, (my * CHUNK,), (CHUNK,))
    condr = owner_my[None, :] == cs[:, None]
    jr = jnp.cumsum(condr, axis=1, dtype=jnp.int32) - 1
    P = (
        condr[:, :, None] & (jr[:, :, None] == caps[None, None, :])
    ).astype(jnp.float32)

    return _all_to_all_combine(yg, G, P, ys)
